# baseline (speedup 1.0000x reference)
import functools

import jax
import jax.numpy as jnp
from jax import lax
from jax.experimental import pallas as pl
from jax.experimental.pallas import tpu as pltpu
from jax.experimental.pallas import tpu_sc as plsc

_NUM_CORES = 2
_NUM_SUBCORES = 16
_NUM_WORKERS = _NUM_CORES * _NUM_SUBCORES
_CHUNK = 128
_LANES = 16


def _transpose_block(src_ref, dst_ref, n_feat, dst_row0, col0, rows_base):
  cols = jnp.full((_LANES,), 0, jnp.int32)
  ones = jnp.full((_LANES,), 1, jnp.int32)
  for d in range(n_feat):
    row_ref = dst_ref.at[dst_row0 + d]
    for bv in range(_CHUNK // _LANES):
      vec = plsc.load_gather(src_ref, [rows_base[bv], cols])
      row_ref[pl.ds(col0 + bv * _LANES, _LANES)] = vec
    cols = cols + ones


def _latent_body(b_per_w, n_chunks, content_dim, class_dim,
                 img_hbm, cls_hbm, ctab_hbm, ktab_hbm, out_hbm,
                 iidx_v, cidx_v, crow_v, krow_v, lat_v,
                 csem, ksem, wsem):
  n_feat = content_dim + class_dim
  wid = lax.axis_index("s") * _NUM_CORES + lax.axis_index("c")
  base = wid * b_per_w
  pltpu.sync_copy(img_hbm.at[pl.ds(base, b_per_w)], iidx_v)
  pltpu.sync_copy(cls_hbm.at[pl.ds(base, b_per_w)], cidx_v)

  iota = lax.iota(jnp.int32, _LANES)
  rows_base = [iota + bv * _LANES for bv in range(_CHUNK // _LANES)]

  def fire(c, par):
    sl = pl.ds(c * _CHUNK, _CHUNK)
    pltpu.async_copy(ctab_hbm.at[iidx_v.at[sl]], crow_v.at[par], csem.at[par])
    pltpu.async_copy(ktab_hbm.at[cidx_v.at[sl]], krow_v.at[par], ksem.at[par])

  fire(0, 0)

  def step(c, carry):
    par = lax.rem(c, 2)
    nxt = lax.rem(c + 1, 2)

    @pl.when(c < n_chunks - 1)
    def _():
      fire(c + 1, nxt)

    sl = pl.ds(c * _CHUNK, _CHUNK)
    pltpu.make_async_copy(ctab_hbm.at[iidx_v.at[sl]], crow_v.at[par],
                          csem.at[par]).wait()
    pltpu.make_async_copy(ktab_hbm.at[cidx_v.at[sl]], krow_v.at[par],
                          ksem.at[par]).wait()

    @pl.when(c >= 2)
    def _():
      pltpu.make_async_copy(
          lat_v.at[par], out_hbm.at[pl.ds(0, n_feat), pl.ds(base, _CHUNK)],
          wsem.at[par]).wait()

    _transpose_block(crow_v.at[par], lat_v.at[par], content_dim, 0, 0,
                     rows_base)
    _transpose_block(krow_v.at[par], lat_v.at[par], class_dim, content_dim,
                     0, rows_base)
    pltpu.async_copy(
        lat_v.at[par],
        out_hbm.at[pl.ds(0, n_feat), pl.ds(base + c * _CHUNK, _CHUNK)],
        wsem.at[par])
    return carry

  lax.fori_loop(0, n_chunks, step, 0)
  for par in range(2):
    pltpu.make_async_copy(
        lat_v.at[par], out_hbm.at[pl.ds(0, n_feat), pl.ds(base, _CHUNK)],
        wsem.at[par]).wait()


def kernel(img_id, class_id, content_table, class_table):
  batch = img_id.shape[0]
  content_dim = content_table.shape[1]
  class_dim = class_table.shape[1]
  n_feat = content_dim + class_dim
  assert batch % (_NUM_WORKERS * _CHUNK) == 0
  b_per_w = batch // _NUM_WORKERS
  n_chunks = b_per_w // _CHUNK

  ktab_padded = jnp.pad(class_table, ((0, 0), (0, content_dim - class_dim)))

  mesh = plsc.VectorSubcoreMesh(core_axis_name="c", subcore_axis_name="s")
  body = functools.partial(_latent_body, b_per_w, n_chunks, content_dim,
                           class_dim)
  latent_t = pl.kernel(
      body,
      out_type=jax.ShapeDtypeStruct((n_feat, batch), jnp.float32),
      mesh=mesh,
      compiler_params=pltpu.CompilerParams(use_tc_tiling_on_sc=True,
                                           needs_layout_passes=False),
      scratch_types=[
          pltpu.VMEM((b_per_w,), jnp.int32),
          pltpu.VMEM((b_per_w,), jnp.int32),
          pltpu.VMEM((2, _CHUNK, content_dim), jnp.float32),
          pltpu.VMEM((2, _CHUNK, content_dim), jnp.float32),
          pltpu.VMEM((2, n_feat, _CHUNK), jnp.float32),
          pltpu.SemaphoreType.DMA((2,)),
          pltpu.SemaphoreType.DMA((2,)),
          pltpu.SemaphoreType.DMA((2,)),
      ],
  )
  out_t = latent_t(img_id, class_id, content_table, ktab_padded)
  return out_t.T


# speedup vs baseline: 1.2563x; 1.2563x over previous
import functools

import jax
import jax.numpy as jnp
from jax import lax
from jax.experimental import pallas as pl
from jax.experimental.pallas import tpu as pltpu
from jax.experimental.pallas import tpu_sc as plsc

_NUM_CORES = 2
_NUM_SUBCORES = 16
_NUM_WORKERS = _NUM_CORES * _NUM_SUBCORES
_CHUNK = 128
_LANES = 16


def _transpose_chunk(crow_ref, krow_ref, dst_ref, content_dim, class_dim):
  iota = lax.iota(jnp.int32, _LANES)
  c_groups = [iota + g * _LANES for g in range(content_dim // _LANES)]
  k_groups = [iota + content_dim + g * _LANES
              for g in range(class_dim // _LANES)]
  col = jnp.full((_LANES,), 0, jnp.int32)
  one = jnp.full((_LANES,), 1, jnp.int32)
  for b in range(_CHUNK):
    c_row = crow_ref.at[b]
    k_row = krow_ref.at[b]
    c_vecs = [c_row[pl.ds(g * _LANES, _LANES)]
              for g in range(content_dim // _LANES)]
    k_vecs = [k_row[pl.ds(g * _LANES, _LANES)]
              for g in range(class_dim // _LANES)]
    for rows, vec in zip(c_groups, c_vecs):
      plsc.store_scatter(dst_ref, [rows, col], vec)
    for rows, vec in zip(k_groups, k_vecs):
      plsc.store_scatter(dst_ref, [rows, col], vec)
    col = col + one


def _latent_body(b_per_w, n_chunks, content_dim, class_dim,
                 img_hbm, cls_hbm, ctab_hbm, ktab_hbm, out_hbm,
                 iidx_v, cidx_v, crow_v, krow_v, lat_v,
                 csem, ksem, wsem):
  n_feat = content_dim + class_dim
  wid = lax.axis_index("s") * _NUM_CORES + lax.axis_index("c")
  base = wid * b_per_w
  pltpu.sync_copy(img_hbm.at[pl.ds(base, b_per_w)], iidx_v)
  pltpu.sync_copy(cls_hbm.at[pl.ds(base, b_per_w)], cidx_v)

  def fire(c, par):
    sl = pl.ds(c * _CHUNK, _CHUNK)
    pltpu.async_copy(ctab_hbm.at[iidx_v.at[sl]], crow_v.at[par], csem.at[par])
    pltpu.async_copy(ktab_hbm.at[cidx_v.at[sl]], krow_v.at[par], ksem.at[par])

  fire(0, 0)

  def step(c, carry):
    par = lax.rem(c, 2)
    nxt = lax.rem(c + 1, 2)

    @pl.when(c < n_chunks - 1)
    def _():
      fire(c + 1, nxt)

    sl = pl.ds(c * _CHUNK, _CHUNK)
    pltpu.make_async_copy(ctab_hbm.at[iidx_v.at[sl]], crow_v.at[par],
                          csem.at[par]).wait()
    pltpu.make_async_copy(ktab_hbm.at[cidx_v.at[sl]], krow_v.at[par],
                          ksem.at[par]).wait()

    @pl.when(c >= 2)
    def _():
      pltpu.make_async_copy(
          lat_v.at[par], out_hbm.at[pl.ds(0, n_feat), pl.ds(base, _CHUNK)],
          wsem.at[par]).wait()

    _transpose_chunk(crow_v.at[par], krow_v.at[par], lat_v.at[par],
                     content_dim, class_dim)
    pltpu.async_copy(
        lat_v.at[par],
        out_hbm.at[pl.ds(0, n_feat), pl.ds(base + c * _CHUNK, _CHUNK)],
        wsem.at[par])
    return carry

  lax.fori_loop(0, n_chunks, step, 0)
  for par in range(2):
    pltpu.make_async_copy(
        lat_v.at[par], out_hbm.at[pl.ds(0, n_feat), pl.ds(base, _CHUNK)],
        wsem.at[par]).wait()


def kernel(img_id, class_id, content_table, class_table):
  batch = img_id.shape[0]
  content_dim = content_table.shape[1]
  class_dim = class_table.shape[1]
  n_feat = content_dim + class_dim
  assert batch % (_NUM_WORKERS * _CHUNK) == 0
  b_per_w = batch // _NUM_WORKERS
  n_chunks = b_per_w // _CHUNK

  ktab_padded = jnp.pad(class_table, ((0, 0), (0, content_dim - class_dim)))

  mesh = plsc.VectorSubcoreMesh(core_axis_name="c", subcore_axis_name="s")
  body = functools.partial(_latent_body, b_per_w, n_chunks, content_dim,
                           class_dim)
  latent_t = pl.kernel(
      body,
      out_type=jax.ShapeDtypeStruct((n_feat, batch), jnp.float32),
      mesh=mesh,
      compiler_params=pltpu.CompilerParams(use_tc_tiling_on_sc=True,
                                           needs_layout_passes=False),
      scratch_types=[
          pltpu.VMEM((b_per_w,), jnp.int32),
          pltpu.VMEM((b_per_w,), jnp.int32),
          pltpu.VMEM((2, _CHUNK, content_dim), jnp.float32),
          pltpu.VMEM((2, _CHUNK, content_dim), jnp.float32),
          pltpu.VMEM((2, n_feat, _CHUNK), jnp.float32),
          pltpu.SemaphoreType.DMA((2,)),
          pltpu.SemaphoreType.DMA((2,)),
          pltpu.SemaphoreType.DMA((2,)),
      ],
  )
  out_t = latent_t(img_id, class_id, content_table, ktab_padded)
  return out_t.T


# speedup vs baseline: 2.3655x; 1.8829x over previous
import functools

import jax
import jax.numpy as jnp
from jax import lax
from jax.experimental import pallas as pl
from jax.experimental.pallas import tpu as pltpu
from jax.experimental.pallas import tpu_sc as plsc

_NUM_CORES = 2
_NUM_SUBCORES = 16
_NUM_WORKERS = _NUM_CORES * _NUM_SUBCORES
_CHUNK = 128
_LANES = 16


def _transpose_chunk(crow_ref, krow_ref, dst_ref, content_dim, class_dim):
  iota = lax.iota(jnp.int32, _LANES)

  def k_step(k, carry):
    rot = lax.rem(iota + k, _LANES)
    for d0 in range(0, content_dim, _LANES):
      f = rot + d0
      for b0 in range(0, _CHUNK, _LANES):
        rows = iota + b0
        vec = plsc.load_gather(crow_ref, [rows, f])
        plsc.store_scatter(dst_ref, [f, rows], vec)
    for d0 in range(0, class_dim, _LANES):
      f = rot + d0
      fd = f + content_dim
      for b0 in range(0, _CHUNK, _LANES):
        rows = iota + b0
        vec = plsc.load_gather(krow_ref, [rows, f])
        plsc.store_scatter(dst_ref, [fd, rows], vec)
    return carry

  lax.fori_loop(0, _LANES, k_step, 0)


def _latent_body(b_per_w, n_chunks, content_dim, class_dim,
                 img_hbm, cls_hbm, ctab_hbm, ktab_hbm, out_hbm,
                 iidx_v, cidx_v, crow_v, krow_v, lat_v,
                 csem, ksem, wsem):
  n_feat = content_dim + class_dim
  wid = lax.axis_index("s") * _NUM_CORES + lax.axis_index("c")
  base = wid * b_per_w
  pltpu.sync_copy(img_hbm.at[pl.ds(base, b_per_w)], iidx_v)
  pltpu.sync_copy(cls_hbm.at[pl.ds(base, b_per_w)], cidx_v)

  def fire(c, par):
    sl = pl.ds(c * _CHUNK, _CHUNK)
    pltpu.async_copy(ctab_hbm.at[iidx_v.at[sl]], crow_v.at[par], csem.at[par])
    pltpu.async_copy(ktab_hbm.at[cidx_v.at[sl]], krow_v.at[par], ksem.at[par])

  fire(0, 0)

  def step(c, carry):
    par = lax.rem(c, 2)
    nxt = lax.rem(c + 1, 2)

    @pl.when(c < n_chunks - 1)
    def _():
      fire(c + 1, nxt)

    sl = pl.ds(c * _CHUNK, _CHUNK)
    pltpu.make_async_copy(ctab_hbm.at[iidx_v.at[sl]], crow_v.at[par],
                          csem.at[par]).wait()
    pltpu.make_async_copy(ktab_hbm.at[cidx_v.at[sl]], krow_v.at[par],
                          ksem.at[par]).wait()

    @pl.when(c >= 2)
    def _():
      pltpu.make_async_copy(
          lat_v.at[par], out_hbm.at[pl.ds(0, n_feat), pl.ds(base, _CHUNK)],
          wsem.at[par]).wait()

    _transpose_chunk(crow_v.at[par], krow_v.at[par], lat_v.at[par],
                     content_dim, class_dim)
    pltpu.async_copy(
        lat_v.at[par],
        out_hbm.at[pl.ds(0, n_feat), pl.ds(base + c * _CHUNK, _CHUNK)],
        wsem.at[par])
    return carry

  lax.fori_loop(0, n_chunks, step, 0)
  for par in range(2):
    pltpu.make_async_copy(
        lat_v.at[par], out_hbm.at[pl.ds(0, n_feat), pl.ds(base, _CHUNK)],
        wsem.at[par]).wait()


def kernel(img_id, class_id, content_table, class_table):
  batch = img_id.shape[0]
  content_dim = content_table.shape[1]
  class_dim = class_table.shape[1]
  n_feat = content_dim + class_dim
  assert batch % (_NUM_WORKERS * _CHUNK) == 0
  b_per_w = batch // _NUM_WORKERS
  n_chunks = b_per_w // _CHUNK

  ktab_padded = jnp.pad(class_table, ((0, 0), (0, content_dim - class_dim)))

  mesh = plsc.VectorSubcoreMesh(core_axis_name="c", subcore_axis_name="s")
  body = functools.partial(_latent_body, b_per_w, n_chunks, content_dim,
                           class_dim)
  latent_t = pl.kernel(
      body,
      out_type=jax.ShapeDtypeStruct((n_feat, batch), jnp.float32),
      mesh=mesh,
      compiler_params=pltpu.CompilerParams(use_tc_tiling_on_sc=True,
                                           needs_layout_passes=False),
      scratch_types=[
          pltpu.VMEM((b_per_w,), jnp.int32),
          pltpu.VMEM((b_per_w,), jnp.int32),
          pltpu.VMEM((2, _CHUNK, content_dim), jnp.float32),
          pltpu.VMEM((2, _CHUNK, content_dim), jnp.float32),
          pltpu.VMEM((2, n_feat, _CHUNK), jnp.float32),
          pltpu.SemaphoreType.DMA((2,)),
          pltpu.SemaphoreType.DMA((2,)),
          pltpu.SemaphoreType.DMA((2,)),
      ],
  )
  out_t = latent_t(img_id, class_id, content_table, ktab_padded)
  return out_t.T
